# baseline (speedup 1.0000x reference)
import jax
import jax.numpy as jnp
from jax import lax
from jax.experimental import pallas as pl
from jax.experimental.pallas import tpu as pltpu
from jax.experimental.pallas import tpu_sc as plsc

VOCAB = 1000000
D_MODEL = 64
BATCH = 4096
SEQ = 200

NC = 2
NS = 16
NW = NC * NS

TCOLS = VOCAB // 128
TAIL = VOCAB - TCOLS * 128

_MESH = dict(core_axis_name="c", subcore_axis_name="s")


def _wid():
    return lax.axis_index("s") * NC + lax.axis_index("c")


def _repack_body(wt_hbm, tail_hbm, tbl_hbm, src, dst, isem, osem):
    wid = _wid()
    iota = lax.iota(jnp.int32, 16)
    rows_half = lax.shift_right_logical(iota, 1)
    colpar = (iota & 1) * 64

    NBLK = TCOLS // 4
    nblks = (NBLK - wid + NW - 1) // NW
    ntot = (NBLK + NW - 1) // NW

    def fire_in(m, b):
        bk = wid + NW * m
        pltpu.async_copy(wt_hbm.at[:, pl.ds(bk * 512, 512)], src.at[b], isem)

    def wait_in(b):
        pltpu.make_async_copy(
            wt_hbm.at[:, pl.ds(0, 512)], src.at[b], isem
        ).wait()

    def fire_out(m, b, sub):
        k = (wid + NW * m) * 4 + sub
        pltpu.async_copy(dst.at[b, sub], tbl_hbm.at[pl.ds(k * 64, 64)], osem)

    def wait_out(b):
        for sub in range(4):
            pltpu.make_async_copy(
                dst.at[b, 0], tbl_hbm.at[pl.ds(0, 64)], osem
            ).wait()

    @pl.when(0 < nblks)
    def _():
        fire_in(0, 0)

    @pl.loop(0, ntot, step=2)
    def _(m0):
        for b in range(2):
            m = m0 + b

            @pl.when(m < nblks)
            def _():
                @pl.when(m + 1 < nblks)
                def _():
                    fire_in(m + 1, 1 - b)
                wait_in(b)

                @pl.when(m >= 2)
                def _():
                    wait_out(b)

                for sub in range(4):
                    @pl.loop(0, D_MODEL, unroll=4)
                    def _(d):
                        cols = colpar + d
                        vals = [
                            src[b, d, pl.ds(sub * 128 + l0 * 16, 16)]
                            for l0 in range(8)
                        ]
                        for l0 in range(8):
                            plsc.store_scatter(
                                dst.at[b, sub],
                                [rows_half + (l0 * 8), cols],
                                vals[l0],
                            )
                    fire_out(m, b, sub)

    @pl.when(nblks >= 2)
    def _():
        wait_out(0)
        wait_out(1)

    @pl.when(wid == NW - 1)
    def _():
        pltpu.sync_copy(tail_hbm, dst.at[0, 0, pl.ds(0, 32)])
        pltpu.sync_copy(
            dst.at[0, 0, pl.ds(0, 32)], tbl_hbm.at[pl.ds(TCOLS * 64, 32)]
        )


def _repack(wt, tail2d):
    run = pl.kernel(
        _repack_body,
        out_type=jax.ShapeDtypeStruct((VOCAB // 2, 128), jnp.float32),
        mesh=plsc.VectorSubcoreMesh(**_MESH),
        scratch_types=[
            pltpu.VMEM((2, D_MODEL, 512), jnp.float32),
            pltpu.VMEM((2, 4, D_MODEL, 128), jnp.float32),
            pltpu.SemaphoreType.DMA,
            pltpu.SemaphoreType.DMA,
        ],
        compiler_params=pltpu.CompilerParams(
            use_tc_tiling_on_sc=True, needs_layout_passes=False
        ),
    )
    return run(wt, tail2d)


UNITS_PER_W = (SEQ // 8) * 32 // NW


def _lookup_body(tokt_hbm, tbl_hbm, out_hbm, idx_v, gath, trans, gsem0, gsem1, osem):
    wid = _wid()
    iota = lax.iota(jnp.int32, 16)
    dflat = [mm * 16 + iota for mm in range(4)]
    gsems = [gsem0, gsem1]

    def wait_wb(tb):
        for i in range(8):
            pltpu.make_async_copy(
                trans.at[tb, pl.ds(0, 8), pl.ds(0, 128)],
                out_hbm.at[0, 0, 0],
                osem,
            ).wait()

    @pl.loop(0, UNITS_PER_W)
    def _(n):
        g = wid * UNITS_PER_W + n
        sblk = g // 32
        j = g - 32 * sblk
        s0 = sblk * 8
        pltpu.sync_copy(
            tokt_hbm.at[pl.ds(s0, 8), pl.ds(j * 128, 128)], idx_v
        )

        def fire_row(r8, b):
            pltpu.async_copy(tbl_hbm.at[idx_v.at[r8]], gath.at[b], gsems[b])

        def wait_row(b):
            pltpu.make_async_copy(
                tbl_hbm.at[idx_v.at[0]], gath.at[b], gsems[b]
            ).wait()

        fire_row(0, 0)
        for r8 in range(8):
            b = r8 % 2
            tb = r8 % 2
            if r8 < 7:
                fire_row(r8 + 1, 1 - b)
            wait_row(b)
            if r8 >= 2:
                wait_wb(tb)
            else:
                @pl.when(n > 0)
                def _():
                    wait_wb(tb)

            @pl.loop(0, 128, unroll=8)
            def _(t):
                colt = jnp.broadcast_to(t, (16,)).astype(jnp.int32)
                vals = [gath[b, t, pl.ds(mm * 16, 16)] for mm in range(4)]
                for mm in range(4):
                    plsc.store_scatter(
                        trans.at[tb], [dflat[mm], colt], vals[mm]
                    )
            for i in range(8):
                pltpu.async_copy(
                    trans.at[tb, pl.ds(8 * i, 8), pl.ds(0, 128)],
                    out_hbm.at[s0 + r8, i, j],
                    osem,
                )

    wait_wb(0)
    wait_wb(1)


def _lookup(tokt, tbl):
    run = pl.kernel(
        _lookup_body,
        out_type=jax.ShapeDtypeStruct((SEQ, 8, 32, 8, 128), jnp.float32),
        mesh=plsc.VectorSubcoreMesh(**_MESH),
        scratch_types=[
            pltpu.VMEM((8, 128), jnp.int32),
            pltpu.VMEM((2, 128, D_MODEL), jnp.float32),
            pltpu.VMEM((2, D_MODEL, 131), jnp.float32),
            pltpu.SemaphoreType.DMA,
            pltpu.SemaphoreType.DMA,
            pltpu.SemaphoreType.DMA,
        ],
        compiler_params=pltpu.CompilerParams(
            use_tc_tiling_on_sc=False, needs_layout_passes=False
        ),
    )
    return run(tokt, tbl)


def kernel(token_ids, weight):
    wt = weight.T
    tail2d = weight[TCOLS * 128:].reshape(32, 128)
    tbl = _repack(wt, tail2d).reshape(VOCAB, D_MODEL)
    tokt = token_ids.T.astype(jnp.int32)
    out5 = _lookup(tokt, tbl)
    return out5.transpose(2, 4, 0, 1, 3).reshape(BATCH, SEQ, D_MODEL)


# speedup vs baseline: 1.6746x; 1.6746x over previous
import jax
import jax.numpy as jnp
from jax import lax
from jax.experimental import pallas as pl
from jax.experimental.pallas import tpu as pltpu
from jax.experimental.pallas import tpu_sc as plsc

VOCAB = 1000000
D_MODEL = 64
BATCH = 4096
SEQ = 200

NC = 2
NS = 16
NW = NC * NS

TCOLS = VOCAB // 128
TAIL = VOCAB - TCOLS * 128

_MESH = dict(core_axis_name="c", subcore_axis_name="s")


def _wid():
    return lax.axis_index("s") * NC + lax.axis_index("c")


def _repack_body(wt_hbm, tail_hbm, tbl_hbm, src, dst, isem, osem):
    wid = _wid()
    iota = lax.iota(jnp.int32, 16)
    rows_half = lax.shift_right_logical(iota, 1)
    colpar = (iota & 1) * 64

    NBLK = TCOLS // 4
    nblks = (NBLK - wid + NW - 1) // NW
    ntot = (NBLK + NW - 1) // NW

    def fire_in(m, b):
        bk = wid + NW * m
        pltpu.async_copy(wt_hbm.at[:, pl.ds(bk * 512, 512)], src.at[b], isem)

    def wait_in(b):
        pltpu.make_async_copy(
            wt_hbm.at[:, pl.ds(0, 512)], src.at[b], isem
        ).wait()

    def fire_out(m, b, sub):
        k = (wid + NW * m) * 4 + sub
        pltpu.async_copy(dst.at[b, sub], tbl_hbm.at[pl.ds(k * 64, 64)], osem)

    def wait_out(b):
        for sub in range(4):
            pltpu.make_async_copy(
                dst.at[b, 0], tbl_hbm.at[pl.ds(0, 64)], osem
            ).wait()

    @pl.when(0 < nblks)
    def _():
        fire_in(0, 0)

    @pl.loop(0, ntot, step=2)
    def _(m0):
        for b in range(2):
            m = m0 + b

            @pl.when(m < nblks)
            def _():
                @pl.when(m + 1 < nblks)
                def _():
                    fire_in(m + 1, 1 - b)
                wait_in(b)

                @pl.when(m >= 2)
                def _():
                    wait_out(b)

                for sub in range(4):
                    @pl.loop(0, D_MODEL, unroll=4)
                    def _(d):
                        cols = colpar + d
                        vals = [
                            src[b, d, pl.ds(sub * 128 + l0 * 16, 16)]
                            for l0 in range(8)
                        ]
                        for l0 in range(8):
                            plsc.store_scatter(
                                dst.at[b, sub],
                                [rows_half + (l0 * 8), cols],
                                vals[l0],
                            )
                    fire_out(m, b, sub)

    @pl.when(nblks >= 2)
    def _():
        wait_out(0)
        wait_out(1)

    @pl.when(wid == NW - 1)
    def _():
        pltpu.sync_copy(tail_hbm, dst.at[0, 0, pl.ds(0, 32)])
        pltpu.sync_copy(
            dst.at[0, 0, pl.ds(0, 32)], tbl_hbm.at[pl.ds(TCOLS * 64, 32)]
        )


def _repack(wt, tail2d):
    run = pl.kernel(
        _repack_body,
        out_type=jax.ShapeDtypeStruct((VOCAB // 2, 128), jnp.float32),
        mesh=plsc.VectorSubcoreMesh(**_MESH),
        scratch_types=[
            pltpu.VMEM((2, D_MODEL, 512), jnp.float32),
            pltpu.VMEM((2, 4, D_MODEL, 128), jnp.float32),
            pltpu.SemaphoreType.DMA,
            pltpu.SemaphoreType.DMA,
        ],
        compiler_params=pltpu.CompilerParams(
            use_tc_tiling_on_sc=True, needs_layout_passes=False
        ),
    )
    return run(wt, tail2d)


UNITS_PER_W = (SEQ // 8) * 32 // NW


def _lookup_body(tokt_hbm, tbl_hbm, out_hbm, idx_v, gath, trans, gsem0, gsem1, osem):
    wid = _wid()
    iota = lax.iota(jnp.int32, 16)
    dflat = [mm * 16 + iota for mm in range(4)]
    gsems = [gsem0, gsem1]

    def wait_wb(tb):
        for i in range(8):
            pltpu.make_async_copy(
                trans.at[tb, pl.ds(0, 8), pl.ds(0, 128)],
                out_hbm.at[0, 0, 0],
                osem,
            ).wait()

    @pl.loop(0, UNITS_PER_W)
    def _(n):
        g = wid * UNITS_PER_W + n
        sblk = g // 32
        j = g - 32 * sblk
        s0 = sblk * 8
        pltpu.sync_copy(
            tokt_hbm.at[pl.ds(s0, 8), pl.ds(j * 128, 128)], idx_v
        )

        def fire_row(r8, b):
            pltpu.async_copy(tbl_hbm.at[idx_v.at[r8]], gath.at[b], gsems[b])

        def wait_row(b):
            pltpu.make_async_copy(
                tbl_hbm.at[idx_v.at[0]], gath.at[b], gsems[b]
            ).wait()

        fire_row(0, 0)
        for r8 in range(8):
            b = r8 % 2
            tb = r8 % 2
            if r8 < 7:
                fire_row(r8 + 1, 1 - b)
            wait_row(b)
            if r8 >= 2:
                wait_wb(tb)
            else:
                @pl.when(n > 0)
                def _():
                    wait_wb(tb)

            @pl.loop(0, 128, unroll=8)
            def _(t):
                colt = jnp.broadcast_to(t, (16,)).astype(jnp.int32)
                vals = [gath[b, t, pl.ds(mm * 16, 16)] for mm in range(4)]
                for mm in range(4):
                    plsc.store_scatter(
                        trans.at[tb], [dflat[mm], colt], vals[mm]
                    )
            for i in range(8):
                pltpu.async_copy(
                    trans.at[tb, pl.ds(8 * i, 8), pl.ds(0, 128)],
                    out_hbm.at[s0 + r8, i, j],
                    osem,
                )

    wait_wb(0)
    wait_wb(1)


def _lookup(tokt, tbl):
    run = pl.kernel(
        _lookup_body,
        out_type=jax.ShapeDtypeStruct((SEQ, 8, 32, 8, 128), jnp.float32),
        mesh=plsc.VectorSubcoreMesh(**_MESH),
        scratch_types=[
            pltpu.VMEM((8, 128), jnp.int32),
            pltpu.VMEM((2, 128, D_MODEL), jnp.float32),
            pltpu.VMEM((2, D_MODEL, 131), jnp.float32),
            pltpu.SemaphoreType.DMA,
            pltpu.SemaphoreType.DMA,
            pltpu.SemaphoreType.DMA,
        ],
        compiler_params=pltpu.CompilerParams(
            use_tc_tiling_on_sc=False, needs_layout_passes=False
        ),
    )
    return run(tokt, tbl)


def kernel(token_ids, weight):
    tokt = token_ids.T.astype(jnp.int32)
    out5 = _lookup(tokt, weight)
    return out5.transpose(2, 4, 0, 1, 3).reshape(BATCH, SEQ, D_MODEL)
